# baseline (speedup 1.0000x reference)
import functools
import math

import jax
import jax.numpy as jnp
from jax.experimental import pallas as pl
from jax.experimental.pallas import tpu as pltpu

QUANT = 0.9
TA = 512
TB = 256


def _fused_kernel(
    na, nb, k_th,
    x_ref, wo_ref, bo_ref, wp_ref, bp_ref,
    o_ref,
    y_sc, key_sc, cum_sc,
):
    i = pl.program_id(0)

    @pl.when(i < na)
    def _phase_a():
        y_blk = jnp.dot(
            x_ref[...], wo_ref[...], preferred_element_type=jnp.float32
        ) + bo_ref[...][None, :]
        y_sc[:, pl.ds(i * TA, TA)] = y_blk

    @pl.when(i >= na)
    def _phase_b():
        j = i - na
        z_blk = jnp.dot(
            y_sc[...], wp_ref[...], preferred_element_type=jnp.float32
        ) + bp_ref[...][None, :]
        u = jax.lax.bitcast_convert_type(z_blk, jnp.int32)
        keys = jnp.where(u >= 0, u, jnp.int32(-(2**31)) - u)
        key_sc[:, pl.ds(j * TB, TB)] = keys
        cols = []
        for b in range(64):
            t_b = ((b - 31) << 26) - 1 if b < 63 else 2**31 - 1
            cols.append(
                jnp.sum(
                    (keys <= jnp.int32(t_b)).astype(jnp.int32),
                    axis=1,
                    keepdims=True,
                )
            )
        blk_cum = jnp.concatenate(cols, axis=1)
        prev = jnp.where(j == 0, jnp.zeros_like(blk_cum), cum_sc[...])
        cum_sc[...] = prev + blk_cum

    @pl.when(i == na + nb - 1)
    def _epilogue():
        nb_less = jnp.sum(
            (cum_sc[...] < k_th).astype(jnp.int32), axis=1, keepdims=True
        )
        lo = (nb_less - 32) << 26
        hi = lo + (1 << 26) - 1

        def body(_, carry):
            lo, hi = carry
            mid = (lo >> 1) + (hi >> 1) + (lo & hi & 1)
            cnt = jnp.sum(
                (key_sc[...] <= mid).astype(jnp.int32), axis=1, keepdims=True
            )
            ge = cnt >= k_th
            lo = jnp.where(ge, lo, mid + 1)
            hi = jnp.where(ge, mid, hi)
            return lo, hi

        lo, hi = jax.lax.fori_loop(0, 26, body, (lo, hi))
        o_ref[...] = jnp.where(key_sc[...] > lo, y_sc[...], 0.0)


@jax.jit
def kernel(x, W_orig, b_orig, W_pol, b_pol):
    B, D = x.shape
    F = W_pol.shape[1]
    k_th = int(max(1, min(F, 1 + math.floor(QUANT * (F - 1)))))
    na = F // TA
    nb = F // TB

    return pl.pallas_call(
        functools.partial(_fused_kernel, na, nb, k_th),
        grid=(na + nb,),
        in_specs=[
            pl.BlockSpec((B, D), lambda i: (0, 0)),
            pl.BlockSpec((D, TA), lambda i: (0, jnp.minimum(i, na - 1))),
            pl.BlockSpec((TA,), lambda i: (jnp.minimum(i, na - 1),)),
            pl.BlockSpec((F, TB), lambda i: (0, jnp.maximum(0, i - na))),
            pl.BlockSpec((TB,), lambda i: (jnp.maximum(0, i - na),)),
        ],
        out_specs=pl.BlockSpec((B, F), lambda i: (0, 0)),
        out_shape=jax.ShapeDtypeStruct((B, F), jnp.float32),
        scratch_shapes=[
            pltpu.VMEM((B, F), jnp.float32),
            pltpu.VMEM((B, F), jnp.int32),
            pltpu.VMEM((B, 64), jnp.int32),
        ],
        compiler_params=pltpu.CompilerParams(
            dimension_semantics=("arbitrary",),
        ),
    )(x, W_orig, b_orig, W_pol, b_pol)


# speedup vs baseline: 1.1394x; 1.1394x over previous
import functools
import math

import jax
import jax.numpy as jnp
from jax.experimental import pallas as pl
from jax.experimental.pallas import tpu as pltpu

QUANT = 0.9
KB = 512


def _fused_kernel(
    nk, k_th,
    x_ref, wo_ref, bo_ref, wp_ref, bp_ref,
    o_ref,
    y_sc, z_sc,
):
    i = pl.program_id(0)

    @pl.when(i < nk)
    def _mm1_band():
        y_blk = jnp.dot(
            x_ref[...], wo_ref[...], preferred_element_type=jnp.float32
        ) + bo_ref[...][None, :]
        y_sc[:, pl.ds(i * KB, KB)] = y_blk

    @pl.when(i >= 1)
    def _mm2_band():
        j = i - 1
        acc = jnp.dot(
            y_sc[:, pl.ds(j * KB, KB)],
            wp_ref[...],
            preferred_element_type=jnp.float32,
        )
        prev = jnp.where(j == 0, bp_ref[...][None, :], z_sc[...])
        z_sc[...] = prev + acc

    @pl.when(i == nk)
    def _epilogue():
        B = o_ref.shape[0]
        lo = jnp.full((B, 1), -(2**31), jnp.int32)
        hi = jnp.full((B, 1), 2**31 - 1, jnp.int32)

        def body(_, carry):
            lo, hi = carry
            mid = (lo >> 1) + (hi >> 1) + (lo & hi & 1)
            mu = jnp.where(mid >= 0, mid, jnp.int32(-(2**31)) - mid)
            midf = jax.lax.bitcast_convert_type(mu, jnp.float32)
            cnt = jnp.sum(
                (z_sc[...] <= midf).astype(jnp.int32), axis=1, keepdims=True
            )
            ge = cnt >= k_th
            lo = jnp.where(ge, lo, mid + 1)
            hi = jnp.where(ge, mid, hi)
            return lo, hi

        lo, hi = jax.lax.fori_loop(0, 32, body, (lo, hi))
        tu = jnp.where(lo >= 0, lo, jnp.int32(-(2**31)) - lo)
        thr = jax.lax.bitcast_convert_type(tu, jnp.float32)
        o_ref[...] = jnp.where(z_sc[...] > thr, y_sc[...], 0.0)


@jax.jit
def kernel(x, W_orig, b_orig, W_pol, b_pol):
    B, D = x.shape
    F = W_pol.shape[1]
    k_th = int(max(1, min(F, 1 + math.floor(QUANT * (F - 1)))))
    nk = F // KB

    return pl.pallas_call(
        functools.partial(_fused_kernel, nk, k_th),
        grid=(nk + 1,),
        in_specs=[
            pl.BlockSpec((B, D), lambda i: (0, 0)),
            pl.BlockSpec((D, KB), lambda i: (0, jnp.minimum(i, nk - 1))),
            pl.BlockSpec((KB,), lambda i: (jnp.minimum(i, nk - 1),)),
            pl.BlockSpec((KB, F), lambda i: (jnp.maximum(0, i - 1), 0)),
            pl.BlockSpec((F,), lambda i: (0,)),
        ],
        out_specs=pl.BlockSpec((B, F), lambda i: (0, 0)),
        out_shape=jax.ShapeDtypeStruct((B, F), jnp.float32),
        scratch_shapes=[
            pltpu.VMEM((B, F), jnp.float32),
            pltpu.VMEM((B, F), jnp.float32),
        ],
        compiler_params=pltpu.CompilerParams(
            dimension_semantics=("arbitrary",),
            vmem_limit_bytes=100 * 1024 * 1024,
        ),
    )(x, W_orig, b_orig, W_pol, b_pol)
